# baseline (speedup 1.0000x reference)
import jax
import jax.numpy as jnp
from jax import lax
from jax.experimental import pallas as pl
from jax.experimental.pallas import tpu as pltpu
from jax.experimental.pallas import tpu_sc as plsc

N_EDGES = 160000
N_NODES = 10000
D_FEAT = 256

NC = 2
NS = 16
L = 16

SEG_PER_CORE = N_NODES // NC
SEG_PER_PASS = SEG_PER_CORE // 2
ACC_ROWS = 2560
ACC_PER_TILE = ACC_ROWS // NS
CNT_ROWS = SEG_PER_CORE + 8
CNT_PER_TILE = 320
EDGES_PER_TILE = N_EDGES // NS

BLK = 64
NBLK = (EDGES_PER_TILE + BLK - 1) // BLK
LAST_START = EDGES_PER_TILE - BLK
RING = 4
DEPTH = 3


def _body(y_hbm, emap_hbm, out_hbm, acc_sh, cnt_sh, emap_v, ybuf, idxy_v,
          idxc_v, ones_v, cnt16_v,
          sg0, sg1, sg2, sg3, sy0, sy1, sy2, sy3, sc0, sc1, sc2, sc3):
    c = lax.axis_index("c")
    t = lax.axis_index("s")
    base = c * SEG_PER_CORE
    semg = (sg0, sg1, sg2, sg3)
    semy = (sy0, sy1, sy2, sy3)
    semc = (sc0, sc1, sc2, sc3)

    onev = jnp.ones((L,), jnp.float32)
    zerov = jnp.zeros((L,), jnp.float32)
    for r in range(BLK):
        ones_v[r, :] = onev
    for r in range(L):
        cnt16_v[r, :] = zerov

    zero_v = ybuf.at[pl.ds(32, L)]

    def fill_zero_rows():
        for r in range(L):
            for k in range(D_FEAT // L):
                ybuf[32 + r, pl.ds(k * L, L)] = zerov

    e0 = t * EDGES_PER_TILE
    pltpu.sync_copy(emap_hbm.at[pl.ds(e0, EDGES_PER_TILE)], emap_v)

    cnrows = jnp.minimum(CNT_PER_TILE, CNT_ROWS - t * CNT_PER_TILE)

    def zero_cnt(g, _):
        r = pl.multiple_of(
            t * CNT_PER_TILE + jnp.minimum(g * L, cnrows - L), L)
        pltpu.sync_copy(cnt16_v, cnt_sh.at[pl.ds(r, L)])
        return _

    lax.fori_loop(0, CNT_PER_TILE // L, zero_cnt, None)

    def zero_acc(g, _):
        r = pl.multiple_of(t * ACC_PER_TILE + g * L, L)
        pltpu.sync_copy(zero_v, acc_sh.at[pl.ds(r, L)])
        return _

    def block_start(j):
        return pl.multiple_of(jnp.minimum(j * BLK, LAST_START), L)

    def run_pass(p):
        pbase = base + p * SEG_PER_PASS

        def scan(j, carry):
            nlo, nhi = carry
            s = block_start(j)
            bmin = emap_v[pl.ds(s, L)][0]
            bmax = emap_v[pl.ds(s + BLK - L, L)][L - 1]
            return (nlo + (bmax < pbase).astype(jnp.int32),
                    nhi + (bmin < pbase + SEG_PER_PASS).astype(jnp.int32))

        j_lo, nhi = lax.fori_loop(0, NBLK, scan,
                                  (jnp.int32(0), jnp.int32(0)))
        j_hi = nhi - 1

        def gather(j, b):
            pltpu.async_copy(y_hbm.at[pl.ds(e0 + block_start(j), BLK)],
                             ybuf.at[pl.ds(b * BLK, BLK)], semg[b])

        def wait_slot(b):
            pltpu.make_async_copy(ybuf.at[pl.ds(b * BLK, BLK)],
                                  acc_sh.at[idxy_v.at[b]], semy[b]).wait()
            pltpu.make_async_copy(ones_v, cnt_sh.at[idxc_v.at[b]],
                                  semc[b]).wait()

        def process(j, b):
            jb = block_start(j)
            minpos = j * BLK
            iota = lax.iota(jnp.int32, L)

            @pl.when(j + DEPTH - RING >= j_lo)
            def _():
                wait_slot((b + DEPTH) % RING)

            pltpu.make_async_copy(
                y_hbm.at[pl.ds(e0 + jb, BLK)],
                ybuf.at[pl.ds(b * BLK, BLK)], semg[b]).wait()
            for k in range(BLK // L):
                e = emap_v[pl.ds(jb + k * L, L)]
                pos = jb + k * L + iota
                owned = (e >= pbase) & (e < pbase + SEG_PER_PASS) & (
                    pos >= minpos)
                idxy_v[b, pl.ds(k * L, L)] = jnp.where(
                    owned, e - pbase, SEG_PER_PASS)
                idxc_v[b, pl.ds(k * L, L)] = jnp.where(
                    owned, e - base, SEG_PER_CORE)
            pltpu.async_copy(ybuf.at[pl.ds(b * BLK, BLK)],
                             acc_sh.at[idxy_v.at[b]], semy[b], add=True)
            pltpu.async_copy(ones_v, cnt_sh.at[idxc_v.at[b]], semc[b],
                             add=True)

            g = j + DEPTH

            @pl.when(g <= j_hi)
            def _():
                gather(g, (b + DEPTH) % RING)

        @pl.when(j_lo <= j_hi)
        def _():
            nb = j_hi - j_lo + 1
            gather(j_lo, 0)

            @pl.when(j_lo + 1 <= j_hi)
            def _():
                gather(j_lo + 1, 1)

            @pl.when(j_lo + 2 <= j_hi)
            def _():
                gather(j_lo + 2, 2)

            def outer(i, _):
                jj = j_lo + RING * i
                for b in range(RING):
                    j = jj + b

                    @pl.when(j <= j_hi)
                    def _():
                        process(j, b)

                return _

            lax.fori_loop(0, (nb + RING - 1) // RING, outer, None)

            m = (j_hi - j_lo) % RING
            for b in range(RING):
                @pl.when(m == b)
                def _():
                    wait_slot(b)

        plsc.subcore_barrier()

        fbase = t * ACC_PER_TILE
        nrows = jnp.minimum(ACC_PER_TILE, SEG_PER_PASS - fbase)

        def fin_group(g, _):
            r = pl.multiple_of(fbase + jnp.minimum(g * L, nrows - L), 4)
            pltpu.sync_copy(acc_sh.at[pl.ds(r, L)], ybuf.at[pl.ds(0, L)])
            pltpu.sync_copy(cnt_sh.at[pl.ds(p * SEG_PER_PASS + r, L)],
                            cnt16_v)
            for i in range(L):
                cnt = cnt16_v[i, :]
                rec = 1.0 / jnp.maximum(cnt, 1.0)
                for k in range(D_FEAT // L):
                    ybuf[L + i, pl.ds(k * L, L)] = (
                        ybuf[i, pl.ds(k * L, L)] * rec)
            pltpu.sync_copy(ybuf.at[pl.ds(L, L)],
                            out_hbm.at[pl.ds(pbase + r, L)])
            return _

        lax.fori_loop(0, (nrows + L - 1) // L, fin_group, None)

    fill_zero_rows()
    lax.fori_loop(0, ACC_PER_TILE // L, zero_acc, None)
    plsc.subcore_barrier()
    run_pass(0)
    fill_zero_rows()
    lax.fori_loop(0, ACC_PER_TILE // L, zero_acc, None)
    plsc.subcore_barrier()
    run_pass(1)


@jax.jit
def _pooling(y, emap32):
    mesh = plsc.VectorSubcoreMesh(core_axis_name="c", subcore_axis_name="s")
    f = pl.kernel(
        _body,
        out_type=jax.ShapeDtypeStruct((N_NODES, D_FEAT), jnp.float32),
        mesh=mesh,
        scratch_types=[
            pltpu.VMEM_SHARED((ACC_ROWS, D_FEAT), jnp.float32),
            pltpu.VMEM_SHARED((CNT_ROWS, L), jnp.float32),
            pltpu.VMEM((EDGES_PER_TILE,), jnp.int32),
            pltpu.VMEM((RING * BLK, D_FEAT), jnp.float32),
            pltpu.VMEM((RING, BLK), jnp.int32),
            pltpu.VMEM((RING, BLK), jnp.int32),
            pltpu.VMEM((BLK, L), jnp.float32),
            pltpu.VMEM((L, L), jnp.float32),
        ] + [pltpu.SemaphoreType.DMA] * 12,
        compiler_params=pltpu.CompilerParams(use_tc_tiling_on_sc=False),
    )
    return f(y, emap32)


def kernel(Y, e_map, v_count):
    del v_count
    return _pooling(Y, e_map.astype(jnp.int32))


# speedup vs baseline: 1.2597x; 1.2597x over previous
import jax
import jax.numpy as jnp
from jax import lax
from jax.experimental import pallas as pl
from jax.experimental.pallas import tpu as pltpu
from jax.experimental.pallas import tpu_sc as plsc

N_EDGES = 160000
N_NODES = 10000
D_FEAT = 256

NC = 2
NS = 16
L = 16

SEG_PER_CORE = N_NODES // NC
ACC_ROWS = SEG_PER_CORE + 8
ROWS_PER_TILE = 320
EDGES_PER_TILE = N_EDGES // NS
BLK = 64
NBLK = (EDGES_PER_TILE + BLK - 1) // BLK
LAST_START = EDGES_PER_TILE - BLK
FINB = 32


def _body(y_hbm, emap_hbm, out_hbm, acc_sh, cnt_sh, emap_v, ybuf0, ybuf1,
          idx_v, ones_v, cnt32_v, sem0, sem1, semc0, semc1, semz):
    c = lax.axis_index("c")
    t = lax.axis_index("s")
    base = c * SEG_PER_CORE

    onev = jnp.ones((L,), jnp.float32)
    zerov = jnp.zeros((L,), jnp.float32)
    for r in range(BLK):
        ones_v[r, :] = onev
        for k in range(D_FEAT // L):
            ybuf1[r, pl.ds(k * L, L)] = zerov
    for r in range(2 * L):
        cnt32_v[r, :] = zerov

    zbase = t * ROWS_PER_TILE
    znrows = jnp.minimum(ROWS_PER_TILE, ACC_ROWS - zbase)
    NZB = ROWS_PER_TILE // BLK
    for g in range(NZB):
        r = pl.multiple_of(
            zbase + jnp.minimum(g * BLK, znrows - BLK), L)
        pltpu.async_copy(ybuf1, acc_sh.at[pl.ds(r, BLK)], semz)
    for g in range(ROWS_PER_TILE // L):
        r = pl.multiple_of(zbase + jnp.minimum(g * L, znrows - L), L)
        pltpu.async_copy(cnt32_v.at[pl.ds(0, L)], cnt_sh.at[pl.ds(r, L)],
                         semz)

    e0 = t * EDGES_PER_TILE
    pltpu.sync_copy(emap_hbm.at[pl.ds(e0, EDGES_PER_TILE)], emap_v)

    def scan_blocks(j, carry):
        nlo, nhi = carry
        s = pl.multiple_of(jnp.minimum(j * BLK, LAST_START), L)
        bmin = emap_v[pl.ds(s, L)][0]
        bmax = emap_v[pl.ds(s + BLK - L, L)][L - 1]
        return (nlo + (bmax < base).astype(jnp.int32),
                nhi + (bmin < base + SEG_PER_CORE).astype(jnp.int32))

    j_lo, nhi = lax.fori_loop(
        0, NBLK, scan_blocks, (jnp.int32(0), jnp.int32(0)))
    j_hi = nhi - 1

    def block_start(j):
        return pl.multiple_of(jnp.minimum(j * BLK, LAST_START), L)

    def gather(j, buf, sem):
        pltpu.async_copy(y_hbm.at[pl.ds(e0 + block_start(j), BLK)], buf, sem)

    @pl.when(j_lo <= j_hi)
    def _():
        gather(j_lo, ybuf0, sem0)

    def drain_zero(g, _):
        pltpu.make_async_copy(ybuf1, acc_sh.at[pl.ds(zbase, BLK)],
                              semz).wait()
        pltpu.make_async_copy(cnt32_v.at[pl.ds(0, L)],
                              cnt_sh.at[pl.ds(zbase, L)], semz).wait()
        return _

    lax.fori_loop(0, NZB, drain_zero, None)
    for g in range(ROWS_PER_TILE // L - NZB):
        pltpu.make_async_copy(cnt32_v.at[pl.ds(0, L)],
                              cnt_sh.at[pl.ds(zbase, L)], semz).wait()

    @pl.when(j_lo < j_hi)
    def _():
        gather(j_lo + 1, ybuf1, sem1)

    plsc.subcore_barrier()

    def process(j, b, buf, sem, csem):
        jb = block_start(j)
        minpos = j * BLK
        iota = lax.iota(jnp.int32, L)

        @pl.when(j - j_lo >= 2)
        def _():
            pltpu.make_async_copy(ones_v, cnt_sh.at[idx_v.at[b]],
                                  csem).wait()

        for k in range(BLK // L):
            e = emap_v[pl.ds(jb + k * L, L)]
            pos = jb + k * L + iota
            owned = (e >= base) & (e < base + SEG_PER_CORE) & (pos >= minpos)
            idx_v[b, pl.ds(k * L, L)] = jnp.where(owned, e - base,
                                                  SEG_PER_CORE)
        pltpu.async_copy(ones_v, cnt_sh.at[idx_v.at[b]], csem, add=True)
        pltpu.make_async_copy(
            y_hbm.at[pl.ds(e0 + jb, BLK)], buf, sem).wait()
        pltpu.sync_copy(buf, acc_sh.at[idx_v.at[b]], add=True)

    @pl.when(j_lo <= j_hi)
    def _():
        nb = j_hi - j_lo + 1

        def outer(i, _):
            jj = j_lo + 2 * i
            for b, (buf, sem, csem) in enumerate(
                    ((ybuf0, sem0, semc0), (ybuf1, sem1, semc1))):
                j = jj + b

                @pl.when(j <= j_hi)
                def _():
                    process(j, b, buf, sem, csem)

                    @pl.when(j + 2 <= j_hi)
                    def _():
                        gather(j + 2, buf, sem)

            return _

        lax.fori_loop(0, (nb + 1) // 2, outer, None)

        pltpu.make_async_copy(ones_v, cnt_sh.at[idx_v.at[0]], semc0).wait()

        @pl.when(nb >= 2)
        def _():
            pltpu.make_async_copy(ones_v, cnt_sh.at[idx_v.at[1]],
                                  semc1).wait()

    plsc.subcore_barrier()

    fbase = t * ROWS_PER_TILE
    nrows = jnp.minimum(ROWS_PER_TILE, SEG_PER_CORE - fbase)
    ngroups = (nrows + FINB - 1) // FINB

    def fin_group(g, p, wsem):
        r = pl.multiple_of(fbase + jnp.minimum(g * FINB, nrows - FINB), 8)
        pltpu.sync_copy(acc_sh.at[pl.ds(r, FINB)], ybuf0.at[pl.ds(0, FINB)])
        pltpu.sync_copy(cnt_sh.at[pl.ds(r, FINB)], cnt32_v)

        half = ybuf1.at[pl.ds(p * FINB, FINB)]

        @pl.when(g >= 2)
        def _():
            pltpu.make_async_copy(half, out_hbm.at[pl.ds(base + r, FINB)],
                                  wsem).wait()

        for i in range(FINB):
            cnt = cnt32_v[i, :]
            rec = 1.0 / jnp.maximum(cnt, 1.0)
            for k in range(D_FEAT // L):
                ybuf1[p * FINB + i, pl.ds(k * L, L)] = (
                    ybuf0[i, pl.ds(k * L, L)] * rec)
        pltpu.async_copy(half, out_hbm.at[pl.ds(base + r, FINB)], wsem)

    def fin_outer(i, _):
        for p, wsem in enumerate((semc0, semc1)):
            g = 2 * i + p

            @pl.when(g < ngroups)
            def _():
                fin_group(g, p, wsem)

        return _

    lax.fori_loop(0, (ngroups + 1) // 2, fin_outer, None)
    pltpu.make_async_copy(ybuf1.at[pl.ds(0, FINB)],
                          out_hbm.at[pl.ds(base, FINB)], semc0).wait()
    pltpu.make_async_copy(ybuf1.at[pl.ds(FINB, FINB)],
                          out_hbm.at[pl.ds(base, FINB)], semc1).wait()


@jax.jit
def _pooling(y, emap32):
    mesh = plsc.VectorSubcoreMesh(core_axis_name="c", subcore_axis_name="s")
    f = pl.kernel(
        _body,
        out_type=jax.ShapeDtypeStruct((N_NODES, D_FEAT), jnp.float32),
        mesh=mesh,
        scratch_types=[
            pltpu.VMEM_SHARED((ACC_ROWS, D_FEAT), jnp.float32),
            pltpu.VMEM_SHARED((ACC_ROWS, L), jnp.float32),
            pltpu.VMEM((EDGES_PER_TILE,), jnp.int32),
            pltpu.VMEM((BLK, D_FEAT), jnp.float32),
            pltpu.VMEM((BLK, D_FEAT), jnp.float32),
            pltpu.VMEM((2, BLK), jnp.int32),
            pltpu.VMEM((BLK, L), jnp.float32),
            pltpu.VMEM((2 * L, L), jnp.float32),
            pltpu.SemaphoreType.DMA,
            pltpu.SemaphoreType.DMA,
            pltpu.SemaphoreType.DMA,
            pltpu.SemaphoreType.DMA,
            pltpu.SemaphoreType.DMA,
        ],
        compiler_params=pltpu.CompilerParams(use_tc_tiling_on_sc=False),
    )
    return f(y, emap32)


def kernel(Y, e_map, v_count):
    del v_count
    return _pooling(Y, e_map.astype(jnp.int32))
